# baseline (speedup 1.0000x reference)
import functools

import jax
import jax.numpy as jnp
from jax import lax
from jax.experimental import pallas as pl
from jax.experimental.pallas import tpu as pltpu
from jax.experimental.pallas import tpu_sc as plsc

N = 10000
E = 320000
D = 128
G = 64

CHUNK = 128
NTILES = 32
CPT = 80
STAGE = 40
CPT0 = 80
CPT1 = 2 * CPT - CPT0
EPAD = NTILES * CPT * CHUNK
NPAD = 10112
ZR = 128

@functools.cache
def _make_sc_aggregate():
    mesh = plsc.VectorSubcoreMesh(core_axis_name="c", subcore_axis_name="s")
    return functools.partial(
        pl.kernel,
        out_type=jax.ShapeDtypeStruct((2 * N, D), jnp.float32),
        mesh=mesh,
        scratch_types=[
            pltpu.VMEM((STAGE, CHUNK), jnp.int32),
            pltpu.VMEM((STAGE, CHUNK), jnp.int32),
            pltpu.VMEM((CHUNK, D), jnp.float32),
            pltpu.VMEM((CHUNK, D), jnp.float32),
            pltpu.VMEM_SHARED((NPAD, D), jnp.float32),
            pltpu.SemaphoreType.DMA,
            pltpu.SemaphoreType.DMA,
            pltpu.SemaphoreType.DMA,
            pltpu.SemaphoreType.DMA,
        ],
    )(_sc_aggregate_body)


def _sc_aggregate_body(h_hbm, src_hbm, dst_hbm, z_hbm, out_hbm,
                       srcv, dstv, buf0, buf1, agg, sem0, sem1,
                       ssem0, ssem1):
    cid = lax.axis_index("c")
    tid = lax.axis_index("s")
    wid = cid * 16 + tid

    zbase = tid * (NPAD // 16)
    for i in range(4):
        pltpu.sync_copy(z_hbm, agg.at[pl.ds(zbase + i * ZR, ZR)])
    pltpu.sync_copy(z_hbm.at[pl.ds(0, NPAD // 16 - 4 * ZR)],
                    agg.at[pl.ds(zbase + 4 * ZR, NPAD // 16 - 4 * ZR)])

    plsc.subcore_barrier()

    nrounds = jnp.where(cid == 0, CPT0 // STAGE, CPT1 // STAGE)
    tile_base = jnp.where(cid == 0, tid * CPT0, 16 * CPT0 + tid * CPT1)
    for r in range(CPT0 // STAGE):
      @pl.when(r < nrounds)
      def _():
        start = pl.multiple_of(tile_base + r * STAGE, 8)
        pltpu.sync_copy(src_hbm.at[pl.ds(start, STAGE)],
                        srcv)
        pltpu.sync_copy(dst_hbm.at[pl.ds(start, STAGE)],
                        dstv)

        pltpu.async_copy(h_hbm.at[srcv.at[0]], buf0, sem0)
        pltpu.async_copy(h_hbm.at[srcv.at[1]], buf1, sem1)

        @pl.loop(0, STAGE - 2, step=2)
        def _(j):
            pltpu.make_async_copy(h_hbm.at[srcv.at[0]], buf0, sem0).wait()
            pltpu.async_copy(buf0, agg.at[dstv.at[j]], ssem0, add=True)
            pltpu.make_async_copy(h_hbm.at[srcv.at[0]], buf1, sem1).wait()
            pltpu.async_copy(buf1, agg.at[dstv.at[j + 1]], ssem1, add=True)
            pltpu.make_async_copy(h_hbm.at[srcv.at[0]], buf0, ssem0).wait()
            pltpu.async_copy(h_hbm.at[srcv.at[j + 2]], buf0, sem0)
            pltpu.make_async_copy(h_hbm.at[srcv.at[0]], buf1, ssem1).wait()
            pltpu.async_copy(h_hbm.at[srcv.at[j + 3]], buf1, sem1)

        pltpu.make_async_copy(h_hbm.at[srcv.at[0]], buf0, sem0).wait()
        pltpu.async_copy(buf0, agg.at[dstv.at[STAGE - 2]], ssem0, add=True)
        pltpu.make_async_copy(h_hbm.at[srcv.at[0]], buf1, sem1).wait()
        pltpu.async_copy(buf1, agg.at[dstv.at[STAGE - 1]], ssem1, add=True)
        pltpu.make_async_copy(h_hbm.at[srcv.at[0]], buf0, ssem0).wait()
        pltpu.make_async_copy(h_hbm.at[srcv.at[0]], buf1, ssem1).wait()

    plsc.subcore_barrier()

    @pl.when(tid < 15)
    def _():
        pltpu.sync_copy(agg.at[pl.ds(tid * 624, 624)],
                        out_hbm.at[pl.ds(cid * N + tid * 624, 624)])

    @pl.when(tid == 15)
    def _():
        pltpu.sync_copy(agg.at[pl.ds(15 * 624, 640)],
                        out_hbm.at[pl.ds(cid * N + 15 * 624, 640)])


def _tc_layer_body(h_ref, p0_ref, p1_ref, w1_ref, b1_ref, w2_ref, b2_ref,
                   o_ref):
    z = h_ref[...] + p0_ref[...] + p1_ref[...]
    z = jnp.dot(z, w1_ref[...], preferred_element_type=jnp.float32)
    z = jnp.maximum(z + b1_ref[...], 0.0)
    z = jnp.dot(z, w2_ref[...], preferred_element_type=jnp.float32)
    o_ref[...] = jnp.maximum(z + b2_ref[...], 0.0)


_BLK = 1000


def _tc_layer(h, parts, w1, b1, w2, b2):
    return pl.pallas_call(
        _tc_layer_body,
        grid=(N // _BLK,),
        in_specs=[
            pl.BlockSpec((_BLK, D), lambda i: (i, 0)),
            pl.BlockSpec((_BLK, D), lambda i: (i, 0)),
            pl.BlockSpec((_BLK, D), lambda i: (i + N // _BLK, 0)),
            pl.BlockSpec((D, D), lambda i: (0, 0)),
            pl.BlockSpec((1, D), lambda i: (0, 0)),
            pl.BlockSpec((D, D), lambda i: (0, 0)),
            pl.BlockSpec((1, D), lambda i: (0, 0)),
        ],
        out_specs=pl.BlockSpec((_BLK, D), lambda i: (i, 0)),
        out_shape=jax.ShapeDtypeStruct((N, D), jnp.float32),
    )(h, parts, parts, w1, b1.reshape(1, D), w2, b2.reshape(1, D))


def _tc_readout_body(h_ref, b_ref, bc_ref, wh1_ref, bh1_ref, wh2_ref,
                     bh2_ref, o_ref, sums, counts, maxs):
    i = pl.program_id(0)

    @pl.when(i == 0)
    def _():
        sums[...] = jnp.zeros_like(sums)
        counts[...] = jnp.zeros_like(counts)
        maxs[...] = jnp.full_like(maxs, -3.0e38)

    bidx = b_ref[0]
    gi = lax.broadcasted_iota(jnp.int32, (G, _BLK), 0)
    onehot = (bidx == gi).astype(jnp.float32)
    hb = h_ref[...]
    sums[...] += jnp.dot(onehot, hb, preferred_element_type=jnp.float32)
    counts[...] += jnp.sum(onehot, axis=1, keepdims=True) + jnp.zeros(
        (G, D), jnp.float32)

    glo = b_ref[0, 0, 0]
    ghi = b_ref[0, 0, _BLK - 1]
    for g in range(G):
        @pl.when(jnp.logical_and(g >= glo, g <= ghi))
        def _():
            m = bc_ref[...] == g
            mx = jnp.max(jnp.where(m, hb, -3.0e38), axis=0, keepdims=True)
            maxs[g, :] = jnp.maximum(maxs[g, :], mx[0])

    @pl.when(i == pl.num_programs(0) - 1)
    def _():
        mean = sums[...] / jnp.maximum(counts[...], 1.0)
        wh1 = wh1_ref[...]
        hd = (jnp.dot(mean, wh1[:D], preferred_element_type=jnp.float32)
              + jnp.dot(maxs[...], wh1[D:],
                        preferred_element_type=jnp.float32)
              + bh1_ref[...])
        hd = jnp.maximum(hd, 0.0)
        logits = jnp.dot(hd, wh2_ref[...],
                         preferred_element_type=jnp.float32) + bh2_ref[...]
        o_ref[...] = 1.0 / (1.0 + jnp.exp(-logits))


def _tc_readout(h, bidx3, bcol, wh1, bh1, wh2, bh2):
    return pl.pallas_call(
        _tc_readout_body,
        grid=(N // _BLK,),
        in_specs=[
            pl.BlockSpec((_BLK, D), lambda i: (i, 0)),
            pl.BlockSpec((1, 1, _BLK), lambda i: (i, 0, 0)),
            pl.BlockSpec((_BLK, 1), lambda i: (i, 0)),
            pl.BlockSpec((2 * D, D), lambda i: (0, 0)),
            pl.BlockSpec((1, D), lambda i: (0, 0)),
            pl.BlockSpec((D, 1), lambda i: (0, 0)),
            pl.BlockSpec((1, 1), lambda i: (0, 0)),
        ],
        out_specs=pl.BlockSpec((G, 1), lambda i: (0, 0)),
        out_shape=jax.ShapeDtypeStruct((G, 1), jnp.float32),
        scratch_shapes=[
            pltpu.VMEM((G, D), jnp.float32),
            pltpu.VMEM((G, D), jnp.float32),
            pltpu.VMEM((G, D), jnp.float32),
        ],
    )(h, bidx3, bcol, wh1, bh1.reshape(1, D), wh2, bh2.reshape(1, 1))


def kernel(x, edge_index, batch_idx,
           W1_0, b1_0, W2_0, b2_0,
           W1_1, b1_1, W2_1, b2_1,
           W1_2, b1_2, W2_2, b2_2,
           Wh1, bh1, Wh2, bh2):
    src = edge_index[0]
    dst = edge_index[1]
    pad = EPAD - E
    src_junk = jnp.arange(pad, dtype=jnp.int32) * 41 % N
    src_p = jnp.concatenate([src, src_junk]).reshape(NTILES * CPT, CHUNK)
    junk = N + jnp.arange(pad, dtype=jnp.int32) % (NPAD - N)
    dst_p = jnp.concatenate([dst, junk]).reshape(NTILES * CPT, CHUNK)
    z = jnp.zeros((ZR, D), jnp.float32)
    bidx3 = batch_idx.reshape(N // _BLK, 1, _BLK)
    bcol = batch_idx.reshape(N, 1)

    h = x
    for (w1, b1, w2, b2) in ((W1_0, b1_0, W2_0, b2_0),
                             (W1_1, b1_1, W2_1, b2_1),
                             (W1_2, b1_2, W2_2, b2_2)):
        parts = _make_sc_aggregate()(h, src_p, dst_p, z)
        h = _tc_layer(h, parts, w1, b1, w2, b2)

    return _tc_readout(h, bidx3, bcol, Wh1, bh1, Wh2, bh2)


# speedup vs baseline: 1.0907x; 1.0907x over previous
import functools

import jax
import jax.numpy as jnp
from jax import lax
from jax.experimental import pallas as pl
from jax.experimental.pallas import tpu as pltpu
from jax.experimental.pallas import tpu_sc as plsc

N = 10000
E = 320000
D = 128
G = 64

CHUNK = 128
NTILES = 32
CPT = 80
STAGE = 40
EPAD = NTILES * CPT * CHUNK
NPAD = 10112
ZR = 128

@functools.cache
def _make_sc_aggregate():
    mesh = plsc.VectorSubcoreMesh(core_axis_name="c", subcore_axis_name="s")
    return functools.partial(
        pl.kernel,
        out_type=jax.ShapeDtypeStruct((2 * N, D), jnp.float32),
        mesh=mesh,
        scratch_types=[
            pltpu.VMEM((STAGE, CHUNK), jnp.int32),
            pltpu.VMEM((STAGE, CHUNK), jnp.int32),
            pltpu.VMEM((CHUNK, D), jnp.float32),
            pltpu.VMEM((CHUNK, D), jnp.float32),
            pltpu.VMEM_SHARED((NPAD, D), jnp.float32),
            pltpu.SemaphoreType.DMA,
            pltpu.SemaphoreType.DMA,
        ],
    )(_sc_aggregate_body)


def _sc_aggregate_body(h_hbm, src_hbm, dst_hbm, z_hbm, out_hbm,
                       srcv, dstv, buf0, buf1, agg, sem0, sem1):
    cid = lax.axis_index("c")
    tid = lax.axis_index("s")
    wid = cid * 16 + tid
    tile_base = wid * CPT

    def stage_and_prime(r):
        start = pl.multiple_of(tile_base + r * STAGE, 8)
        pltpu.sync_copy(src_hbm.at[pl.ds(start, STAGE)], srcv)
        pltpu.sync_copy(dst_hbm.at[pl.ds(start, STAGE)], dstv)
        pltpu.async_copy(h_hbm.at[srcv.at[0]], buf0, sem0)

    def agg_round():
        @pl.loop(0, STAGE - 2, step=2)
        def _(j):
            pltpu.make_async_copy(h_hbm.at[srcv.at[0]], buf0, sem0).wait()
            pltpu.async_copy(h_hbm.at[srcv.at[j + 1]], buf1, sem1)
            pltpu.sync_copy(buf0, agg.at[dstv.at[j]], add=True)
            pltpu.make_async_copy(h_hbm.at[srcv.at[0]], buf1, sem1).wait()
            pltpu.async_copy(h_hbm.at[srcv.at[j + 2]], buf0, sem0)
            pltpu.sync_copy(buf1, agg.at[dstv.at[j + 1]], add=True)

        pltpu.make_async_copy(h_hbm.at[srcv.at[0]], buf0, sem0).wait()
        pltpu.async_copy(h_hbm.at[srcv.at[STAGE - 1]], buf1, sem1)
        pltpu.sync_copy(buf0, agg.at[dstv.at[STAGE - 2]], add=True)
        pltpu.make_async_copy(h_hbm.at[srcv.at[0]], buf1, sem1).wait()
        pltpu.sync_copy(buf1, agg.at[dstv.at[STAGE - 1]], add=True)

    stage_and_prime(0)
    zbase = tid * (NPAD // 16)
    for i in range(4):
        pltpu.sync_copy(z_hbm, agg.at[pl.ds(zbase + i * ZR, ZR)])
    pltpu.sync_copy(z_hbm.at[pl.ds(0, NPAD // 16 - 4 * ZR)],
                    agg.at[pl.ds(zbase + 4 * ZR, NPAD // 16 - 4 * ZR)])

    plsc.subcore_barrier()

    agg_round()
    stage_and_prime(1)
    agg_round()

    plsc.subcore_barrier()

    @pl.when(tid < 15)
    def _():
        pltpu.sync_copy(agg.at[pl.ds(tid * 624, 624)],
                        out_hbm.at[pl.ds(cid * N + tid * 624, 624)])

    @pl.when(tid == 15)
    def _():
        pltpu.sync_copy(agg.at[pl.ds(15 * 624, 640)],
                        out_hbm.at[pl.ds(cid * N + 15 * 624, 640)])


def _tc_layer_body(h_ref, p0_ref, p1_ref, w1_ref, b1_ref, w2_ref, b2_ref,
                   o_ref):
    z = h_ref[...] + p0_ref[...] + p1_ref[...]
    z = jnp.dot(z, w1_ref[...], preferred_element_type=jnp.float32)
    z = jnp.maximum(z + b1_ref[...], 0.0)
    z = jnp.dot(z, w2_ref[...], preferred_element_type=jnp.float32)
    o_ref[...] = jnp.maximum(z + b2_ref[...], 0.0)


_BLK = 1000


def _tc_layer(h, parts, w1, b1, w2, b2):
    return pl.pallas_call(
        _tc_layer_body,
        grid=(N // _BLK,),
        in_specs=[
            pl.BlockSpec((_BLK, D), lambda i: (i, 0)),
            pl.BlockSpec((_BLK, D), lambda i: (i, 0)),
            pl.BlockSpec((_BLK, D), lambda i: (i + N // _BLK, 0)),
            pl.BlockSpec((D, D), lambda i: (0, 0)),
            pl.BlockSpec((1, D), lambda i: (0, 0)),
            pl.BlockSpec((D, D), lambda i: (0, 0)),
            pl.BlockSpec((1, D), lambda i: (0, 0)),
        ],
        out_specs=pl.BlockSpec((_BLK, D), lambda i: (i, 0)),
        out_shape=jax.ShapeDtypeStruct((N, D), jnp.float32),
    )(h, parts, parts, w1, b1.reshape(1, D), w2, b2.reshape(1, D))


def _tc_readout_body(h_ref, b_ref, bc_ref, wh1_ref, bh1_ref, wh2_ref,
                     bh2_ref, o_ref, sums, counts, maxs):
    i = pl.program_id(0)

    @pl.when(i == 0)
    def _():
        sums[...] = jnp.zeros_like(sums)
        counts[...] = jnp.zeros_like(counts)
        maxs[...] = jnp.full_like(maxs, -3.0e38)

    bidx = b_ref[0]
    gi = lax.broadcasted_iota(jnp.int32, (G, _BLK), 0)
    onehot = (bidx == gi).astype(jnp.float32)
    hb = h_ref[...]
    sums[...] += jnp.dot(onehot, hb, preferred_element_type=jnp.float32)
    counts[...] += jnp.sum(onehot, axis=1, keepdims=True) + jnp.zeros(
        (G, D), jnp.float32)

    glo = b_ref[0, 0, 0]
    ghi = b_ref[0, 0, _BLK - 1]
    for g in range(G):
        @pl.when(jnp.logical_and(g >= glo, g <= ghi))
        def _():
            m = bc_ref[...] == g
            mx = jnp.max(jnp.where(m, hb, -3.0e38), axis=0, keepdims=True)
            maxs[g, :] = jnp.maximum(maxs[g, :], mx[0])

    @pl.when(i == pl.num_programs(0) - 1)
    def _():
        mean = sums[...] / jnp.maximum(counts[...], 1.0)
        wh1 = wh1_ref[...]
        hd = (jnp.dot(mean, wh1[:D], preferred_element_type=jnp.float32)
              + jnp.dot(maxs[...], wh1[D:],
                        preferred_element_type=jnp.float32)
              + bh1_ref[...])
        hd = jnp.maximum(hd, 0.0)
        logits = jnp.dot(hd, wh2_ref[...],
                         preferred_element_type=jnp.float32) + bh2_ref[...]
        o_ref[...] = 1.0 / (1.0 + jnp.exp(-logits))


def _tc_readout(h, bidx3, bcol, wh1, bh1, wh2, bh2):
    return pl.pallas_call(
        _tc_readout_body,
        grid=(N // _BLK,),
        in_specs=[
            pl.BlockSpec((_BLK, D), lambda i: (i, 0)),
            pl.BlockSpec((1, 1, _BLK), lambda i: (i, 0, 0)),
            pl.BlockSpec((_BLK, 1), lambda i: (i, 0)),
            pl.BlockSpec((2 * D, D), lambda i: (0, 0)),
            pl.BlockSpec((1, D), lambda i: (0, 0)),
            pl.BlockSpec((D, 1), lambda i: (0, 0)),
            pl.BlockSpec((1, 1), lambda i: (0, 0)),
        ],
        out_specs=pl.BlockSpec((G, 1), lambda i: (0, 0)),
        out_shape=jax.ShapeDtypeStruct((G, 1), jnp.float32),
        scratch_shapes=[
            pltpu.VMEM((G, D), jnp.float32),
            pltpu.VMEM((G, D), jnp.float32),
            pltpu.VMEM((G, D), jnp.float32),
        ],
    )(h, bidx3, bcol, wh1, bh1.reshape(1, D), wh2, bh2.reshape(1, 1))


def kernel(x, edge_index, batch_idx,
           W1_0, b1_0, W2_0, b2_0,
           W1_1, b1_1, W2_1, b2_1,
           W1_2, b1_2, W2_2, b2_2,
           Wh1, bh1, Wh2, bh2):
    src = edge_index[0]
    dst = edge_index[1]
    pad = EPAD - E
    src_junk = jnp.arange(pad, dtype=jnp.int32) * 41 % N
    src_p = jnp.concatenate([src, src_junk]).reshape(NTILES * CPT, CHUNK)
    junk = N + jnp.arange(pad, dtype=jnp.int32) % (NPAD - N)
    dst_p = jnp.concatenate([dst, junk]).reshape(NTILES * CPT, CHUNK)
    z = jnp.zeros((ZR, D), jnp.float32)
    bidx3 = batch_idx.reshape(N // _BLK, 1, _BLK)
    bcol = batch_idx.reshape(N, 1)

    h = x
    for (w1, b1, w2, b2) in ((W1_0, b1_0, W2_0, b2_0),
                             (W1_1, b1_1, W2_1, b2_1),
                             (W1_2, b1_2, W2_2, b2_2)):
        parts = _make_sc_aggregate()(h, src_p, dst_p, z)
        h = _tc_layer(h, parts, w1, b1, w2, b2)

    return _tc_readout(h, bidx3, bcol, Wh1, bh1, Wh2, bh2)


# speedup vs baseline: 1.3029x; 1.1945x over previous
import functools

import jax
import jax.numpy as jnp
from jax import lax
from jax.experimental import pallas as pl
from jax.experimental.pallas import tpu as pltpu
from jax.experimental.pallas import tpu_sc as plsc

N = 10000
E = 320000
D = 128
G = 64

CHUNK = 64
NTILES = 32
CPT = 160
STAGE = 40
NBUF = 4
EPAD = NTILES * CPT * CHUNK
NPAD = 10112
ZR = 128

@functools.cache
def _make_sc_aggregate():
    mesh = plsc.VectorSubcoreMesh(core_axis_name="c", subcore_axis_name="s")
    return functools.partial(
        pl.kernel,
        out_type=jax.ShapeDtypeStruct((2 * N, D), jnp.float32),
        mesh=mesh,
        scratch_types=[
            pltpu.VMEM((STAGE, CHUNK), jnp.int32),
            pltpu.VMEM((STAGE, CHUNK), jnp.int32),
            pltpu.VMEM((CHUNK, D), jnp.float32),
            pltpu.VMEM((CHUNK, D), jnp.float32),
            pltpu.VMEM((CHUNK, D), jnp.float32),
            pltpu.VMEM((CHUNK, D), jnp.float32),
            pltpu.VMEM_SHARED((NPAD, D), jnp.float32),
            pltpu.SemaphoreType.DMA,
            pltpu.SemaphoreType.DMA,
            pltpu.SemaphoreType.DMA,
            pltpu.SemaphoreType.DMA,
        ],
    )(_sc_aggregate_body)


def _sc_aggregate_body(h_hbm, src_hbm, dst_hbm, z_hbm, out_hbm,
                       srcv, dstv, buf0, buf1, buf2, buf3, agg,
                       sem0, sem1, sem2, sem3):
    cid = lax.axis_index("c")
    tid = lax.axis_index("s")
    wid = cid * 16 + tid
    tile_base = wid * CPT
    bufs = (buf0, buf1, buf2, buf3)
    sems = (sem0, sem1, sem2, sem3)

    def stage_and_prime(r):
        start = pl.multiple_of(tile_base + r * STAGE, 8)
        pltpu.sync_copy(src_hbm.at[pl.ds(start, STAGE)], srcv)
        pltpu.sync_copy(dst_hbm.at[pl.ds(start, STAGE)], dstv)
        for b in range(NBUF):
            pltpu.async_copy(h_hbm.at[srcv.at[b]], bufs[b], sems[b])

    def agg_round():
        @pl.loop(0, STAGE - NBUF, step=NBUF)
        def _(j):
            for b in range(NBUF):
                pltpu.make_async_copy(h_hbm.at[srcv.at[0]], bufs[b],
                                      sems[b]).wait()
                pltpu.sync_copy(bufs[b], agg.at[dstv.at[j + b]], add=True)
                pltpu.async_copy(h_hbm.at[srcv.at[j + NBUF + b]], bufs[b],
                                 sems[b])
        for b in range(NBUF):
            pltpu.make_async_copy(h_hbm.at[srcv.at[0]], bufs[b],
                                  sems[b]).wait()
            pltpu.sync_copy(bufs[b], agg.at[dstv.at[STAGE - NBUF + b]],
                            add=True)

    stage_and_prime(0)
    zbase = tid * (NPAD // 16)
    for i in range(4):
        pltpu.sync_copy(z_hbm, agg.at[pl.ds(zbase + i * ZR, ZR)])
    pltpu.sync_copy(z_hbm.at[pl.ds(0, NPAD // 16 - 4 * ZR)],
                    agg.at[pl.ds(zbase + 4 * ZR, NPAD // 16 - 4 * ZR)])

    plsc.subcore_barrier()

    for r in range(1, CPT // STAGE):
        agg_round()
        stage_and_prime(r)
    agg_round()

    plsc.subcore_barrier()

    @pl.when(tid < 15)
    def _():
        pltpu.sync_copy(agg.at[pl.ds(tid * 624, 624)],
                        out_hbm.at[pl.ds(cid * N + tid * 624, 624)])

    @pl.when(tid == 15)
    def _():
        pltpu.sync_copy(agg.at[pl.ds(15 * 624, 640)],
                        out_hbm.at[pl.ds(cid * N + 15 * 624, 640)])


def _tc_layer_body(h_ref, p0_ref, p1_ref, w1_ref, b1_ref, w2_ref, b2_ref,
                   o_ref):
    z = h_ref[...] + p0_ref[...] + p1_ref[...]
    z = jnp.dot(z, w1_ref[...], preferred_element_type=jnp.float32)
    z = jnp.maximum(z + b1_ref[...], 0.0)
    z = jnp.dot(z, w2_ref[...], preferred_element_type=jnp.float32)
    o_ref[...] = jnp.maximum(z + b2_ref[...], 0.0)


_BLK = 1000


def _tc_layer(h, parts, w1, b1, w2, b2):
    return pl.pallas_call(
        _tc_layer_body,
        grid=(N // _BLK,),
        in_specs=[
            pl.BlockSpec((_BLK, D), lambda i: (i, 0)),
            pl.BlockSpec((_BLK, D), lambda i: (i, 0)),
            pl.BlockSpec((_BLK, D), lambda i: (i + N // _BLK, 0)),
            pl.BlockSpec((D, D), lambda i: (0, 0)),
            pl.BlockSpec((1, D), lambda i: (0, 0)),
            pl.BlockSpec((D, D), lambda i: (0, 0)),
            pl.BlockSpec((1, D), lambda i: (0, 0)),
        ],
        out_specs=pl.BlockSpec((_BLK, D), lambda i: (i, 0)),
        out_shape=jax.ShapeDtypeStruct((N, D), jnp.float32),
    )(h, parts, parts, w1, b1.reshape(1, D), w2, b2.reshape(1, D))


def _tc_readout_body(h_ref, b_ref, bc_ref, wh1_ref, bh1_ref, wh2_ref,
                     bh2_ref, o_ref, sums, counts, maxs):
    i = pl.program_id(0)

    @pl.when(i == 0)
    def _():
        sums[...] = jnp.zeros_like(sums)
        counts[...] = jnp.zeros_like(counts)
        maxs[...] = jnp.full_like(maxs, -3.0e38)

    bidx = b_ref[0]
    gi = lax.broadcasted_iota(jnp.int32, (G, _BLK), 0)
    onehot = (bidx == gi).astype(jnp.float32)
    hb = h_ref[...]
    sums[...] += jnp.dot(onehot, hb, preferred_element_type=jnp.float32)
    counts[...] += jnp.sum(onehot, axis=1, keepdims=True) + jnp.zeros(
        (G, D), jnp.float32)

    glo = b_ref[0, 0, 0]
    ghi = b_ref[0, 0, _BLK - 1]
    for g in range(G):
        @pl.when(jnp.logical_and(g >= glo, g <= ghi))
        def _():
            m = bc_ref[...] == g
            mx = jnp.max(jnp.where(m, hb, -3.0e38), axis=0, keepdims=True)
            maxs[g, :] = jnp.maximum(maxs[g, :], mx[0])

    @pl.when(i == pl.num_programs(0) - 1)
    def _():
        mean = sums[...] / jnp.maximum(counts[...], 1.0)
        wh1 = wh1_ref[...]
        hd = (jnp.dot(mean, wh1[:D], preferred_element_type=jnp.float32)
              + jnp.dot(maxs[...], wh1[D:],
                        preferred_element_type=jnp.float32)
              + bh1_ref[...])
        hd = jnp.maximum(hd, 0.0)
        logits = jnp.dot(hd, wh2_ref[...],
                         preferred_element_type=jnp.float32) + bh2_ref[...]
        o_ref[...] = 1.0 / (1.0 + jnp.exp(-logits))


def _tc_readout(h, bidx3, bcol, wh1, bh1, wh2, bh2):
    return pl.pallas_call(
        _tc_readout_body,
        grid=(N // _BLK,),
        in_specs=[
            pl.BlockSpec((_BLK, D), lambda i: (i, 0)),
            pl.BlockSpec((1, 1, _BLK), lambda i: (i, 0, 0)),
            pl.BlockSpec((_BLK, 1), lambda i: (i, 0)),
            pl.BlockSpec((2 * D, D), lambda i: (0, 0)),
            pl.BlockSpec((1, D), lambda i: (0, 0)),
            pl.BlockSpec((D, 1), lambda i: (0, 0)),
            pl.BlockSpec((1, 1), lambda i: (0, 0)),
        ],
        out_specs=pl.BlockSpec((G, 1), lambda i: (0, 0)),
        out_shape=jax.ShapeDtypeStruct((G, 1), jnp.float32),
        scratch_shapes=[
            pltpu.VMEM((G, D), jnp.float32),
            pltpu.VMEM((G, D), jnp.float32),
            pltpu.VMEM((G, D), jnp.float32),
        ],
    )(h, bidx3, bcol, wh1, bh1.reshape(1, D), wh2, bh2.reshape(1, 1))


def kernel(x, edge_index, batch_idx,
           W1_0, b1_0, W2_0, b2_0,
           W1_1, b1_1, W2_1, b2_1,
           W1_2, b1_2, W2_2, b2_2,
           Wh1, bh1, Wh2, bh2):
    src = edge_index[0]
    dst = edge_index[1]
    pad = EPAD - E
    src_junk = jnp.arange(pad, dtype=jnp.int32) * 41 % N
    src_p = jnp.concatenate([src, src_junk]).reshape(NTILES * CPT, CHUNK)
    junk = N + jnp.arange(pad, dtype=jnp.int32) % (NPAD - N)
    dst_p = jnp.concatenate([dst, junk]).reshape(NTILES * CPT, CHUNK)
    z = jnp.zeros((ZR, D), jnp.float32)
    bidx3 = batch_idx.reshape(N // _BLK, 1, _BLK)
    bcol = batch_idx.reshape(N, 1)

    h = x
    for (w1, b1, w2, b2) in ((W1_0, b1_0, W2_0, b2_0),
                             (W1_1, b1_1, W2_1, b2_1),
                             (W1_2, b1_2, W2_2, b2_2)):
        parts = _make_sc_aggregate()(h, src_p, dst_p, z)
        h = _tc_layer(h, parts, w1, b1, w2, b2)

    return _tc_readout(h, bidx3, bcol, Wh1, bh1, Wh2, bh2)


# speedup vs baseline: 1.3141x; 1.0086x over previous
import functools

import jax
import jax.numpy as jnp
from jax import lax
from jax.experimental import pallas as pl
from jax.experimental.pallas import tpu as pltpu
from jax.experimental.pallas import tpu_sc as plsc

N = 10000
E = 320000
D = 128
G = 64

CHUNK = 64
NTILES = 32
NCHUNKS = E // CHUNK
CPT = 160
LASTC = NCHUNKS - 31 * CPT
STAGE = 40
NBUF = 4
NPAD = 10112
ZR = 128
assert LASTC == STAGE

@functools.cache
def _make_sc_aggregate():
    mesh = plsc.VectorSubcoreMesh(core_axis_name="c", subcore_axis_name="s")
    return functools.partial(
        pl.kernel,
        out_type=jax.ShapeDtypeStruct((2 * N, D), jnp.float32),
        mesh=mesh,
        scratch_types=[
            pltpu.VMEM((STAGE, CHUNK), jnp.int32),
            pltpu.VMEM((STAGE, CHUNK), jnp.int32),
            pltpu.VMEM((CHUNK, D), jnp.float32),
            pltpu.VMEM((CHUNK, D), jnp.float32),
            pltpu.VMEM((CHUNK, D), jnp.float32),
            pltpu.VMEM((CHUNK, D), jnp.float32),
            pltpu.VMEM_SHARED((NPAD, D), jnp.float32),
            pltpu.SemaphoreType.DMA,
            pltpu.SemaphoreType.DMA,
            pltpu.SemaphoreType.DMA,
            pltpu.SemaphoreType.DMA,
        ],
    )(_sc_aggregate_body)


def _sc_aggregate_body(h_hbm, e_hbm, z_hbm, out_hbm,
                       srcv, dstv, buf0, buf1, buf2, buf3, agg,
                       sem0, sem1, sem2, sem3):
    cid = lax.axis_index("c")
    tid = lax.axis_index("s")
    wid = cid * 16 + tid
    tile_base = wid * CPT
    nrounds = jnp.where(wid == 31, 1, CPT // STAGE)
    bufs = (buf0, buf1, buf2, buf3)
    sems = (sem0, sem1, sem2, sem3)

    def stage_and_prime(r):
        start = pl.multiple_of(tile_base + r * STAGE, 8)
        pltpu.sync_copy(e_hbm.at[0].at[pl.ds(start, STAGE)], srcv)
        pltpu.sync_copy(e_hbm.at[1].at[pl.ds(start, STAGE)], dstv)
        for b in range(NBUF):
            pltpu.async_copy(h_hbm.at[srcv.at[b]], bufs[b], sems[b])

    def agg_round():
        @pl.loop(0, STAGE - NBUF, step=NBUF)
        def _(j):
            for b in range(NBUF):
                pltpu.make_async_copy(h_hbm.at[srcv.at[0]], bufs[b],
                                      sems[b]).wait()
                pltpu.sync_copy(bufs[b], agg.at[dstv.at[j + b]], add=True)
                pltpu.async_copy(h_hbm.at[srcv.at[j + NBUF + b]], bufs[b],
                                 sems[b])
        for b in range(NBUF):
            pltpu.make_async_copy(h_hbm.at[srcv.at[0]], bufs[b],
                                  sems[b]).wait()
            pltpu.sync_copy(bufs[b], agg.at[dstv.at[STAGE - NBUF + b]],
                            add=True)

    stage_and_prime(0)
    zbase = tid * (NPAD // 16)
    for i in range(4):
        pltpu.sync_copy(z_hbm, agg.at[pl.ds(zbase + i * ZR, ZR)])
    pltpu.sync_copy(z_hbm.at[pl.ds(0, NPAD // 16 - 4 * ZR)],
                    agg.at[pl.ds(zbase + 4 * ZR, NPAD // 16 - 4 * ZR)])

    plsc.subcore_barrier()

    for r in range(CPT // STAGE):
        @pl.when(r < nrounds)
        def _():
            agg_round()
        if r + 1 < CPT // STAGE:
            @pl.when(r + 1 < nrounds)
            def _():
                stage_and_prime(r + 1)

    plsc.subcore_barrier()

    @pl.when(tid < 15)
    def _():
        pltpu.sync_copy(agg.at[pl.ds(tid * 624, 624)],
                        out_hbm.at[pl.ds(cid * N + tid * 624, 624)])

    @pl.when(tid == 15)
    def _():
        pltpu.sync_copy(agg.at[pl.ds(15 * 624, 640)],
                        out_hbm.at[pl.ds(cid * N + 15 * 624, 640)])


def _tc_layer_body(h_ref, p0_ref, p1_ref, w1_ref, b1_ref, w2_ref, b2_ref,
                   o_ref):
    z = h_ref[...] + p0_ref[...] + p1_ref[...]
    z = jnp.dot(z, w1_ref[...], preferred_element_type=jnp.float32)
    z = jnp.maximum(z + b1_ref[...], 0.0)
    z = jnp.dot(z, w2_ref[...], preferred_element_type=jnp.float32)
    o_ref[...] = jnp.maximum(z + b2_ref[...], 0.0)


_BLK = 1000


def _tc_layer(h, parts, w1, b1, w2, b2):
    return pl.pallas_call(
        _tc_layer_body,
        grid=(N // _BLK,),
        in_specs=[
            pl.BlockSpec((_BLK, D), lambda i: (i, 0)),
            pl.BlockSpec((_BLK, D), lambda i: (i, 0)),
            pl.BlockSpec((_BLK, D), lambda i: (i + N // _BLK, 0)),
            pl.BlockSpec((D, D), lambda i: (0, 0)),
            pl.BlockSpec((1, D), lambda i: (0, 0)),
            pl.BlockSpec((D, D), lambda i: (0, 0)),
            pl.BlockSpec((1, D), lambda i: (0, 0)),
        ],
        out_specs=pl.BlockSpec((_BLK, D), lambda i: (i, 0)),
        out_shape=jax.ShapeDtypeStruct((N, D), jnp.float32),
    )(h, parts, parts, w1, b1.reshape(1, D), w2, b2.reshape(1, D))


def _tc_readout_body(h_ref, p0_ref, p1_ref, w1_ref, b1_ref, w2_ref, b2_ref,
                     b_ref, bc_ref, wh1_ref, bh1_ref, wh2_ref,
                     bh2_ref, o_ref, sums, counts, maxs):
    i = pl.program_id(0)

    @pl.when(i == 0)
    def _():
        sums[...] = jnp.zeros_like(sums)
        counts[...] = jnp.zeros_like(counts)
        maxs[...] = jnp.full_like(maxs, -3.0e38)

    z = h_ref[...] + p0_ref[...] + p1_ref[...]
    z = jnp.dot(z, w1_ref[...], preferred_element_type=jnp.float32)
    z = jnp.maximum(z + b1_ref[...], 0.0)
    z = jnp.dot(z, w2_ref[...], preferred_element_type=jnp.float32)
    hb = jnp.maximum(z + b2_ref[...], 0.0)

    bidx = b_ref[0]
    gi = lax.broadcasted_iota(jnp.int32, (G, _BLK), 0)
    onehot = (bidx == gi).astype(jnp.float32)
    sums[...] += jnp.dot(onehot, hb, preferred_element_type=jnp.float32)
    counts[...] += jnp.sum(onehot, axis=1, keepdims=True) + jnp.zeros(
        (G, D), jnp.float32)

    glo = b_ref[0, 0, 0]
    ghi = b_ref[0, 0, _BLK - 1]
    for g in range(G):
        @pl.when(jnp.logical_and(g >= glo, g <= ghi))
        def _():
            m = bc_ref[...] == g
            mx = jnp.max(jnp.where(m, hb, -3.0e38), axis=0, keepdims=True)
            maxs[g, :] = jnp.maximum(maxs[g, :], mx[0])

    @pl.when(i == pl.num_programs(0) - 1)
    def _():
        mean = sums[...] / jnp.maximum(counts[...], 1.0)
        wh1 = wh1_ref[...]
        hd = (jnp.dot(mean, wh1[:D], preferred_element_type=jnp.float32)
              + jnp.dot(maxs[...], wh1[D:],
                        preferred_element_type=jnp.float32)
              + bh1_ref[...])
        hd = jnp.maximum(hd, 0.0)
        logits = jnp.dot(hd, wh2_ref[...],
                         preferred_element_type=jnp.float32) + bh2_ref[...]
        o_ref[...] = 1.0 / (1.0 + jnp.exp(-logits))


def _tc_readout(h, parts, w1, b1, w2, b2, bidx3, bcol, wh1, bh1, wh2, bh2):
    return pl.pallas_call(
        _tc_readout_body,
        grid=(N // _BLK,),
        in_specs=[
            pl.BlockSpec((_BLK, D), lambda i: (i, 0)),
            pl.BlockSpec((_BLK, D), lambda i: (i, 0)),
            pl.BlockSpec((_BLK, D), lambda i: (i + N // _BLK, 0)),
            pl.BlockSpec((D, D), lambda i: (0, 0)),
            pl.BlockSpec((1, D), lambda i: (0, 0)),
            pl.BlockSpec((D, D), lambda i: (0, 0)),
            pl.BlockSpec((1, D), lambda i: (0, 0)),
            pl.BlockSpec((1, 1, _BLK), lambda i: (i, 0, 0)),
            pl.BlockSpec((_BLK, 1), lambda i: (i, 0)),
            pl.BlockSpec((2 * D, D), lambda i: (0, 0)),
            pl.BlockSpec((1, D), lambda i: (0, 0)),
            pl.BlockSpec((D, 1), lambda i: (0, 0)),
            pl.BlockSpec((1, 1), lambda i: (0, 0)),
        ],
        out_specs=pl.BlockSpec((G, 1), lambda i: (0, 0)),
        out_shape=jax.ShapeDtypeStruct((G, 1), jnp.float32),
        scratch_shapes=[
            pltpu.VMEM((G, D), jnp.float32),
            pltpu.VMEM((G, D), jnp.float32),
            pltpu.VMEM((G, D), jnp.float32),
        ],
    )(h, parts, parts, w1, b1.reshape(1, D), w2, b2.reshape(1, D),
      bidx3, bcol, wh1, bh1.reshape(1, D), wh2, bh2.reshape(1, 1))


def kernel(x, edge_index, batch_idx,
           W1_0, b1_0, W2_0, b2_0,
           W1_1, b1_1, W2_1, b2_1,
           W1_2, b1_2, W2_2, b2_2,
           Wh1, bh1, Wh2, bh2):
    e3 = edge_index.reshape(2, NCHUNKS, CHUNK)
    z = jnp.zeros((ZR, D), jnp.float32)
    bidx3 = batch_idx.reshape(N // _BLK, 1, _BLK)
    bcol = batch_idx.reshape(N, 1)

    h = x
    for (w1, b1, w2, b2) in ((W1_0, b1_0, W2_0, b2_0),
                             (W1_1, b1_1, W2_1, b2_1)):
        parts = _make_sc_aggregate()(h, e3, z)
        h = _tc_layer(h, parts, w1, b1, w2, b2)

    parts = _make_sc_aggregate()(h, e3, z)
    return _tc_readout(h, parts, W1_2, b1_2, W2_2, b2_2,
                       bidx3, bcol, Wh1, bh1, Wh2, bh2)


# speedup vs baseline: 1.3229x; 1.0068x over previous
import functools

import jax
import jax.numpy as jnp
from jax import lax
from jax.experimental import pallas as pl
from jax.experimental.pallas import tpu as pltpu
from jax.experimental.pallas import tpu_sc as plsc

N = 10000
E = 320000
D = 128
G = 64

CHUNK = 64
NTILES = 32
NCHUNKS = E // CHUNK
CPT = 160
LASTC = NCHUNKS - 31 * CPT
STAGE = 40
NBUF = 4
NPAD = 10112
ZR = 128

@functools.cache
def _make_sc_aggregate():
    mesh = plsc.VectorSubcoreMesh(core_axis_name="c", subcore_axis_name="s")
    return functools.partial(
        pl.kernel,
        out_type=jax.ShapeDtypeStruct((2 * N, D), jnp.float32),
        mesh=mesh,
        scratch_types=[
            pltpu.VMEM((STAGE, CHUNK), jnp.int32),
            pltpu.VMEM((STAGE, CHUNK), jnp.int32),
            pltpu.VMEM((CHUNK, D), jnp.float32),
            pltpu.VMEM((CHUNK, D), jnp.float32),
            pltpu.VMEM((CHUNK, D), jnp.float32),
            pltpu.VMEM((CHUNK, D), jnp.float32),
            pltpu.VMEM_SHARED((NPAD, D), jnp.float32),
            pltpu.SemaphoreType.DMA,
            pltpu.SemaphoreType.DMA,
            pltpu.SemaphoreType.DMA,
            pltpu.SemaphoreType.DMA,
        ],
    )(_sc_aggregate_body)


def _sc_aggregate_body(h_hbm, e_hbm, z_hbm, out_hbm,
                       srcv, dstv, buf0, buf1, buf2, buf3, agg,
                       sem0, sem1, sem2, sem3):
    cid = lax.axis_index("c")
    tid = lax.axis_index("s")
    wid = cid * 16 + tid
    tile_base = wid * CPT
    nrounds = jnp.where(wid == 31, 1, CPT // STAGE)
    nchunks = jnp.where(wid == 31, LASTC, STAGE)
    bufs = (buf0, buf1, buf2, buf3)
    sems = (sem0, sem1, sem2, sem3)

    def stage_and_prime(r):
        start = pl.multiple_of(tile_base + r * STAGE, 8)

        @pl.when(wid < 31)
        def _():
            pltpu.sync_copy(e_hbm.at[0].at[pl.ds(start, STAGE)], srcv)
            pltpu.sync_copy(e_hbm.at[1].at[pl.ds(start, STAGE)], dstv)

        @pl.when(wid == 31)
        def _():
            pltpu.sync_copy(e_hbm.at[0].at[pl.ds(start, LASTC)],
                            srcv.at[pl.ds(0, LASTC)])
            pltpu.sync_copy(e_hbm.at[1].at[pl.ds(start, LASTC)],
                            dstv.at[pl.ds(0, LASTC)])

        for b in range(NBUF):
            pltpu.async_copy(h_hbm.at[srcv.at[b]], bufs[b], sems[b])

    def agg_round():
        @pl.loop(0, nchunks - NBUF, step=NBUF)
        def _(j):
            for b in range(NBUF):
                pltpu.make_async_copy(h_hbm.at[srcv.at[0]], bufs[b],
                                      sems[b]).wait()
                pltpu.sync_copy(bufs[b], agg.at[dstv.at[j + b]], add=True)
                pltpu.async_copy(h_hbm.at[srcv.at[j + NBUF + b]], bufs[b],
                                 sems[b])
        for b in range(NBUF):
            pltpu.make_async_copy(h_hbm.at[srcv.at[0]], bufs[b],
                                  sems[b]).wait()
            pltpu.sync_copy(bufs[b], agg.at[dstv.at[nchunks - NBUF + b]],
                            add=True)

    stage_and_prime(0)
    zbase = tid * (NPAD // 16)
    for i in range(4):
        pltpu.sync_copy(z_hbm, agg.at[pl.ds(zbase + i * ZR, ZR)])
    pltpu.sync_copy(z_hbm.at[pl.ds(0, NPAD // 16 - 4 * ZR)],
                    agg.at[pl.ds(zbase + 4 * ZR, NPAD // 16 - 4 * ZR)])

    plsc.subcore_barrier()

    for r in range(CPT // STAGE):
        @pl.when(r < nrounds)
        def _():
            agg_round()
        if r + 1 < CPT // STAGE:
            @pl.when(r + 1 < nrounds)
            def _():
                stage_and_prime(r + 1)

    plsc.subcore_barrier()

    @pl.when(tid < 15)
    def _():
        pltpu.sync_copy(agg.at[pl.ds(tid * 624, 624)],
                        out_hbm.at[pl.ds(cid * N + tid * 624, 624)])

    @pl.when(tid == 15)
    def _():
        pltpu.sync_copy(agg.at[pl.ds(15 * 624, 640)],
                        out_hbm.at[pl.ds(cid * N + 15 * 624, 640)])


def _tc_layer_body(h_ref, p0_ref, p1_ref, w1_ref, b1_ref, w2_ref, b2_ref,
                   o_ref):
    z = h_ref[...] + p0_ref[...] + p1_ref[...]
    z = jnp.dot(z, w1_ref[...], preferred_element_type=jnp.float32)
    z = jnp.maximum(z + b1_ref[...], 0.0)
    z = jnp.dot(z, w2_ref[...], preferred_element_type=jnp.float32)
    o_ref[...] = jnp.maximum(z + b2_ref[...], 0.0)


_BLK = 2000


def _tc_layer(h, parts, w1, b1, w2, b2):
    return pl.pallas_call(
        _tc_layer_body,
        grid=(N // _BLK,),
        in_specs=[
            pl.BlockSpec((_BLK, D), lambda i: (i, 0)),
            pl.BlockSpec((_BLK, D), lambda i: (i, 0)),
            pl.BlockSpec((_BLK, D), lambda i: (i + N // _BLK, 0)),
            pl.BlockSpec((D, D), lambda i: (0, 0)),
            pl.BlockSpec((1, D), lambda i: (0, 0)),
            pl.BlockSpec((D, D), lambda i: (0, 0)),
            pl.BlockSpec((1, D), lambda i: (0, 0)),
        ],
        out_specs=pl.BlockSpec((_BLK, D), lambda i: (i, 0)),
        out_shape=jax.ShapeDtypeStruct((N, D), jnp.float32),
    )(h, parts, parts, w1, b1.reshape(1, D), w2, b2.reshape(1, D))


def _tc_readout_body(h_ref, p0_ref, p1_ref, w1_ref, b1_ref, w2_ref, b2_ref,
                     b_ref, bc_ref, wh1_ref, bh1_ref, wh2_ref,
                     bh2_ref, o_ref, sums, counts, maxs):
    i = pl.program_id(0)

    @pl.when(i == 0)
    def _():
        sums[...] = jnp.zeros_like(sums)
        counts[...] = jnp.zeros_like(counts)
        maxs[...] = jnp.full_like(maxs, -3.0e38)

    z = h_ref[...] + p0_ref[...] + p1_ref[...]
    z = jnp.dot(z, w1_ref[...], preferred_element_type=jnp.float32)
    z = jnp.maximum(z + b1_ref[...], 0.0)
    z = jnp.dot(z, w2_ref[...], preferred_element_type=jnp.float32)
    hb = jnp.maximum(z + b2_ref[...], 0.0)

    bidx = b_ref[0]
    gi = lax.broadcasted_iota(jnp.int32, (G, _BLK), 0)
    onehot = (bidx == gi).astype(jnp.float32)
    sums[...] += jnp.dot(onehot, hb, preferred_element_type=jnp.float32)
    counts[...] += jnp.sum(onehot, axis=1, keepdims=True) + jnp.zeros(
        (G, D), jnp.float32)

    glo = b_ref[0, 0, 0]
    ghi = b_ref[0, 0, _BLK - 1]
    for g in range(G):
        @pl.when(jnp.logical_and(g >= glo, g <= ghi))
        def _():
            m = bc_ref[...] == g
            mx = jnp.max(jnp.where(m, hb, -3.0e38), axis=0, keepdims=True)
            maxs[g, :] = jnp.maximum(maxs[g, :], mx[0])

    @pl.when(i == pl.num_programs(0) - 1)
    def _():
        mean = sums[...] / jnp.maximum(counts[...], 1.0)
        wh1 = wh1_ref[...]
        hd = (jnp.dot(mean, wh1[:D], preferred_element_type=jnp.float32)
              + jnp.dot(maxs[...], wh1[D:],
                        preferred_element_type=jnp.float32)
              + bh1_ref[...])
        hd = jnp.maximum(hd, 0.0)
        logits = jnp.dot(hd, wh2_ref[...],
                         preferred_element_type=jnp.float32) + bh2_ref[...]
        o_ref[...] = 1.0 / (1.0 + jnp.exp(-logits))


def _tc_readout(h, parts, w1, b1, w2, b2, bidx3, bcol, wh1, bh1, wh2, bh2):
    return pl.pallas_call(
        _tc_readout_body,
        grid=(N // _BLK,),
        in_specs=[
            pl.BlockSpec((_BLK, D), lambda i: (i, 0)),
            pl.BlockSpec((_BLK, D), lambda i: (i, 0)),
            pl.BlockSpec((_BLK, D), lambda i: (i + N // _BLK, 0)),
            pl.BlockSpec((D, D), lambda i: (0, 0)),
            pl.BlockSpec((1, D), lambda i: (0, 0)),
            pl.BlockSpec((D, D), lambda i: (0, 0)),
            pl.BlockSpec((1, D), lambda i: (0, 0)),
            pl.BlockSpec((1, 1, _BLK), lambda i: (i, 0, 0)),
            pl.BlockSpec((_BLK, 1), lambda i: (i, 0)),
            pl.BlockSpec((2 * D, D), lambda i: (0, 0)),
            pl.BlockSpec((1, D), lambda i: (0, 0)),
            pl.BlockSpec((D, 1), lambda i: (0, 0)),
            pl.BlockSpec((1, 1), lambda i: (0, 0)),
        ],
        out_specs=pl.BlockSpec((G, 1), lambda i: (0, 0)),
        out_shape=jax.ShapeDtypeStruct((G, 1), jnp.float32),
        scratch_shapes=[
            pltpu.VMEM((G, D), jnp.float32),
            pltpu.VMEM((G, D), jnp.float32),
            pltpu.VMEM((G, D), jnp.float32),
        ],
    )(h, parts, parts, w1, b1.reshape(1, D), w2, b2.reshape(1, D),
      bidx3, bcol, wh1, bh1.reshape(1, D), wh2, bh2.reshape(1, 1))


def kernel(x, edge_index, batch_idx,
           W1_0, b1_0, W2_0, b2_0,
           W1_1, b1_1, W2_1, b2_1,
           W1_2, b1_2, W2_2, b2_2,
           Wh1, bh1, Wh2, bh2):
    e3 = edge_index.reshape(2, NCHUNKS, CHUNK)
    z = jnp.zeros((ZR, D), jnp.float32)
    bidx3 = batch_idx.reshape(N // _BLK, 1, _BLK)
    bcol = batch_idx.reshape(N, 1)

    h = x
    for (w1, b1, w2, b2) in ((W1_0, b1_0, W2_0, b2_0),
                             (W1_1, b1_1, W2_1, b2_1)):
        parts = _make_sc_aggregate()(h, e3, z)
        h = _tc_layer(h, parts, w1, b1, w2, b2)

    parts = _make_sc_aggregate()(h, e3, z)
    return _tc_readout(h, parts, W1_2, b1_2, W2_2, b2_2,
                       bidx3, bcol, Wh1, bh1, Wh2, bh2)
